# baseline (speedup 1.0000x reference)
import functools

import jax
import jax.numpy as jnp
from jax import lax
from jax.experimental import pallas as pl
from jax.experimental.pallas import tpu as pltpu
from jax.experimental.pallas import tpu_sc as plsc

N = 10000
E = 320000
H = 128
D_OUT = 64
NEG_SLOPE = 0.2

NC = 2
NS = 16
NW = NC * NS
RPT = 320
NPAD = NW * RPT
CAP = 11520
CHUNK = 128
ECHUNK = 4000


def _tc_first_body(x_ref, w_ref, as_ref, ad_ref, hb_ref, als_ref, ald_ref, sc_ref):
    h = jnp.dot(x_ref[...], w_ref[...], preferred_element_type=jnp.float32)
    hb_ref[...] = h.astype(jnp.bfloat16)
    als = jnp.sum(h * as_ref[...], axis=1)
    ald = jnp.sum(h * ad_ref[...], axis=1)
    als_ref[...] = als
    ald_ref[...] = ald
    sc_ref[...] = jnp.full((16,), jnp.max(als), jnp.float32)


def _tc_first(x, w, a_src, a_dst):
    return pl.pallas_call(
        _tc_first_body,
        out_shape=(
            jax.ShapeDtypeStruct((NPAD, H), jnp.bfloat16),
            jax.ShapeDtypeStruct((NPAD,), jnp.float32),
            jax.ShapeDtypeStruct((NPAD,), jnp.float32),
            jax.ShapeDtypeStruct((16,), jnp.float32),
        ),
    )(x, w, a_src.reshape(1, H), a_dst.reshape(1, H))


def _tc_mid_body(acc_ref, b_ref, w_ref, as_ref, ad_ref,
                 hb_ref, als_ref, ald_ref, sc_ref):
    x = jnp.maximum(acc_ref[...] + b_ref[...], 0.0)
    h = jnp.dot(x, w_ref[...], preferred_element_type=jnp.float32)
    hb_ref[...] = h.astype(jnp.bfloat16)
    als = jnp.sum(h * as_ref[...], axis=1)
    ald = jnp.sum(h * ad_ref[...], axis=1)
    als_ref[...] = als
    ald_ref[...] = ald
    sc_ref[...] = jnp.full((16,), jnp.max(als), jnp.float32)


def _tc_mid(acc, b, w, a_src, a_dst):
    return pl.pallas_call(
        _tc_mid_body,
        out_shape=(
            jax.ShapeDtypeStruct((NPAD, H), jnp.bfloat16),
            jax.ShapeDtypeStruct((NPAD,), jnp.float32),
            jax.ShapeDtypeStruct((NPAD,), jnp.float32),
            jax.ShapeDtypeStruct((16,), jnp.float32),
        ),
    )(acc, b.reshape(1, H), w, a_src.reshape(1, H), a_dst.reshape(1, H))


def _tc_final_body(acc_ref, b_ref, w_ref, bl_ref, o_ref):
    x = jnp.maximum(acc_ref[...] + b_ref[...], 0.0)
    out = jnp.dot(x, w_ref[...], preferred_element_type=jnp.float32)
    o_ref[...] = out[:N] + bl_ref[...]


def _tc_final(acc, b, wl, bl):
    return pl.pallas_call(
        _tc_final_body,
        out_shape=jax.ShapeDtypeStruct((N, D_OUT), jnp.float32),
    )(acc, b.reshape(1, H), wl, bl.reshape(1, D_OUT))


def _worker_id():
    return lax.axis_index("s") * NC + lax.axis_index("c")


NBUF_PF = 4


def _prefilter_body(src_hbm, dst_hbm, csrc_hbm, cdst_hbm, cnt_hbm,
                    src_v0, src_v1, src_v2, src_v3,
                    dst_v0, dst_v1, dst_v2, dst_v3, csrc_v, cdst_v, cnt_v,
                    sems_s, sems_d):
    src_v = (src_v0, src_v1, src_v2, src_v3)
    dst_v = (dst_v0, dst_v1, dst_v2, dst_v3)
    wid = _worker_id()
    lo = wid * RPT
    lov = jnp.full((16,), lo, jnp.int32)
    hiv = jnp.full((16,), lo + RPT, jnp.int32)
    trash = lax.iota(jnp.int32, 16) + jnp.full((16,), CAP - 16, jnp.int32)
    nchunks = E // ECHUNK

    def issue(i, b):
        off = i * ECHUNK
        pltpu.async_copy(src_hbm.at[pl.ds(off, ECHUNK)], src_v[b], sems_s.at[b])
        pltpu.async_copy(dst_hbm.at[pl.ds(off, ECHUNK)], dst_v[b], sems_d.at[b])

    def waitb(b):
        pltpu.make_async_copy(src_hbm.at[pl.ds(0, ECHUNK)], src_v[b],
                              sems_s.at[b]).wait()
        pltpu.make_async_copy(dst_hbm.at[pl.ds(0, ECHUNK)], dst_v[b],
                              sems_d.at[b]).wait()

    for b in range(NBUF_PF):
        issue(b, b)

    def grp(s16, d16, n):
        msk = (d16 >= lov) & (d16 < hiv)
        mi = jnp.where(msk, jnp.full((16,), 1, jnp.int32),
                       jnp.full((16,), 0, jnp.int32))
        inc = plsc.cumsum(mi)
        pcv = plsc.all_reduce_population_count(msk)
        nv = jnp.full((16,), n - 1, jnp.int32)
        pos = jnp.where(msk, inc + nv, trash)
        plsc.store_scatter(csrc_v, [pos], s16)
        plsc.store_scatter(cdst_v, [pos], d16 - lov)
        return jnp.minimum(n + pcv[0], CAP - 176)

    def outer(J, n):
        for b in range(NBUF_PF):
            i = J * NBUF_PF + b
            waitb(b)

            def grp2(g, n, b=b):
                n = grp(src_v[b][pl.ds(g * 32, 16)],
                        dst_v[b][pl.ds(g * 32, 16)], n)
                n = grp(src_v[b][pl.ds(g * 32 + 16, 16)],
                        dst_v[b][pl.ds(g * 32 + 16, 16)], n)
                return n

            n = lax.fori_loop(0, ECHUNK // 32, grp2, n)

            @pl.when(i + NBUF_PF < nchunks)
            def _(i=i, b=b):
                issue(i + NBUF_PF, b)
        return n

    n = lax.fori_loop(0, nchunks // NBUF_PF, outer, jnp.int32(0))
    n = jnp.minimum(n, CAP - 176)

    zeros = jnp.zeros((16,), jnp.int32)

    def zb(k, _):
        csrc_v[pl.ds(n + k * 16, 16)] = zeros
        cdst_v[pl.ds(n + k * 16, 16)] = zeros
        return 0

    lax.fori_loop(0, (CHUNK + 16) // 16, zb, 0)

    cnt_v[...] = jnp.full((16,), n, jnp.int32)
    pltpu.sync_copy(csrc_v, csrc_hbm.at[wid])
    pltpu.sync_copy(cdst_v, cdst_hbm.at[wid])
    pltpu.sync_copy(cnt_v, cnt_hbm.at[wid])


@functools.cache
def _prefilter_kernel():
    mesh = plsc.VectorSubcoreMesh(
        core_axis_name="c", subcore_axis_name="s", num_cores=NC, num_subcores=NS)
    return pl.kernel(
        _prefilter_body,
        out_type=(
            jax.ShapeDtypeStruct((NW, CAP), jnp.int32),
            jax.ShapeDtypeStruct((NW, CAP), jnp.int32),
            jax.ShapeDtypeStruct((NW, 16), jnp.int32),
        ),
        mesh=mesh,
        compiler_params=pltpu.CompilerParams(needs_layout_passes=False),
        scratch_types=(
            [pltpu.VMEM((ECHUNK,), jnp.int32)] * (2 * NBUF_PF)
            + [
                pltpu.VMEM((CAP,), jnp.int32),
                pltpu.VMEM((CAP,), jnp.int32),
                pltpu.VMEM((16,), jnp.int32),
                pltpu.SemaphoreType.DMA((NBUF_PF,)),
                pltpu.SemaphoreType.DMA((NBUF_PF,)),
            ]
        ),
    )


NBUF = 5


def _edge_body(csrc_hbm, cdst_hbm, cnt_hbm, h_hbm, als_hbm, ald_hbm, scon_hbm,
               acc_hbm,
               csrc_v, cdst_v, cnt_v, als_v, ald_v, scon_v, rows_v,
               acc_v, den_v, sems):
    wid = _worker_id()
    lo = wid * RPT

    pltpu.sync_copy(csrc_hbm.at[wid], csrc_v)
    pltpu.sync_copy(cdst_hbm.at[wid], cdst_v)
    pltpu.sync_copy(cnt_hbm.at[wid], cnt_v)
    pltpu.sync_copy(als_hbm, als_v)
    pltpu.sync_copy(ald_hbm.at[pl.ds(lo, RPT)], ald_v)
    pltpu.sync_copy(scon_hbm, scon_v)

    zrow = jnp.zeros((16,), jnp.float32)

    def zacc(i, _):
        for u in range(4):
            acc_v[pl.ds((i * 4 + u) * 16, 16)] = zrow
        return 0

    lax.fori_loop(0, RPT * 8 // 4, zacc, 0)

    def zden(i, _):
        den_v[pl.ds(i * 16, 16)] = zrow
        return 0

    lax.fori_loop(0, (RPT + 16) // 16, zden, 0)

    n_e = cnt_v[pl.ds(0, 16)][0]
    nchunks = (n_e + CHUNK - 1) // CHUNK
    Sv = scon_v[pl.ds(0, 16)]
    iota = lax.iota(jnp.int32, 16)
    lane0 = iota == 0
    den_trash = RPT + iota
    cols = [iota + 16 * r for r in range(8)]
    kidxs = [jnp.full((16,), k, jnp.int32) for k in range(16)]
    h128 = jnp.full((16,), H, jnp.int32)
    iota2 = iota * 2
    cols_e = [iota2 + 32 * q for q in range(4)]
    cols_o = [iota2 + 32 * q + 1 for q in range(4)]

    def bcast(v, k):
        return v.at[kidxs[k]].get(mode="promise_in_bounds")

    hw_hbm = h_hbm

    def issue(j, b):
        pltpu.async_copy(
            hw_hbm.at[csrc_v.at[pl.ds(j * CHUNK, CHUNK)]],
            rows_v.at[b], sems.at[b])

    def waitb(b):
        pltpu.make_async_copy(
            hw_hbm.at[csrc_v.at[pl.ds(0, CHUNK)]],
            rows_v.at[b], sems.at[b]).wait()

    for b in range(NBUF):
        @pl.when(b < nchunks)
        def _(b=b):
            issue(b, b)

    def outer(J, _):
        for b in range(NBUF):
            j = J * NBUF + b

            @pl.when(j < nchunks)
            def _(j=j, b=b):
                base = j * CHUNK
                waitb(b)

                def grp_body(g, _):
                    off = base + g * 16
                    s16 = csrc_v[pl.ds(off, 16)]
                    dl16 = cdst_v[pl.ds(off, 16)]
                    sa = plsc.load_gather(als_v, [s16])
                    da = plsc.load_gather(ald_v, [dl16])
                    e = sa + da
                    e = jnp.where(e > 0, e, NEG_SLOPE * e)
                    u = Sv + da
                    m = jnp.where(u > 0, u, NEG_SLOPE * u)
                    t = jnp.exp(e - m)
                    pos = iota + jnp.full((16,), off, jnp.int32)
                    nev = jnp.full((16,), n_e, jnp.int32)
                    t = jnp.where(pos < nev, t, jnp.zeros((16,), jnp.float32))
                    db16 = dl16 * h128
                    ei0 = g * 16
                    for k in range(16):
                        tkv = bcast(t, k)
                        dbv = bcast(db16, k)
                        dnv = bcast(dl16, k)
                        for q in range(4):
                            w16 = rows_v[b, ei0 + k, pl.ds(16 * q, 16)]
                            ab = plsc.bitcast(w16, jnp.bfloat16)
                            ae, ao = plsc.unpack(
                                ab, format=plsc.PackFormat.INTERLEAVED)
                            plsc.addupdate_scatter(
                                acc_v, [dbv + cols_e[q]], ae * tkv)
                            plsc.addupdate_scatter(
                                acc_v, [dbv + cols_o[q]], ao * tkv)
                        didx = jnp.where(lane0, dnv, den_trash)
                        plsc.addupdate_scatter(den_v, [didx], tkv)
                    return 0

                lax.fori_loop(0, CHUNK // 16, grp_body, 0)

                @pl.when(j + NBUF < nchunks)
                def _(j=j, b=b):
                    issue(j + NBUF, b)
        return 0

    lax.fori_loop(0, (nchunks + NBUF - 1) // NBUF, outer, 0)

    def norm(i, _):
        dv = den_v[pl.ds(i * 16, 16)]
        rv = 1.0 / (dv + 1e-16)
        for k in range(16):
            rkv = bcast(rv, k)
            base = (i * 16 + k) * H
            for r in range(8):
                acc_v[pl.ds(base + 16 * r, 16)] = (
                    acc_v[pl.ds(base + 16 * r, 16)] * rkv)
        return 0

    lax.fori_loop(0, RPT // 16, norm, 0)

    pltpu.sync_copy(acc_v, acc_hbm.at[pl.ds(lo * H, RPT * H)])


@functools.cache
def _edge_kernel():
    mesh = plsc.VectorSubcoreMesh(
        core_axis_name="c", subcore_axis_name="s", num_cores=NC, num_subcores=NS)
    return pl.kernel(
        _edge_body,
        out_type=jax.ShapeDtypeStruct((NPAD * H,), jnp.float32),
        mesh=mesh,
        compiler_params=pltpu.CompilerParams(
            needs_layout_passes=False, use_tc_tiling_on_sc=False),
        scratch_types=[
            pltpu.VMEM((CAP,), jnp.int32),
            pltpu.VMEM((CAP,), jnp.int32),
            pltpu.VMEM((16,), jnp.int32),
            pltpu.VMEM((NPAD,), jnp.float32),
            pltpu.VMEM((RPT,), jnp.float32),
            pltpu.VMEM((16,), jnp.float32),
            pltpu.VMEM((NBUF, CHUNK, H // 2), jnp.int32),
            pltpu.VMEM((RPT * H,), jnp.float32),
            pltpu.VMEM((RPT + 16,), jnp.float32),
            pltpu.SemaphoreType.DMA((NBUF,)),
        ],
    )


def kernel(x, edge_index, W1, as1, ad1, b1, W2, as2, ad2, b2, W3, as3, ad3, b3, Wl, bl):
    src = edge_index[0]
    dst = edge_index[1]
    xp = jnp.zeros((NPAD, H), jnp.float32).at[:N].set(x)

    csrc, cdst, cnt = _prefilter_kernel()(src, dst)

    def _as_words(hb):
        return lax.bitcast_convert_type(
            hb.reshape(NPAD, H // 2, 2), jnp.int32)

    hb, als, ald, scon = _tc_first(xp, W1, as1, ad1)
    acc = _edge_kernel()(csrc, cdst, cnt, _as_words(hb), als, ald, scon
                         ).reshape(NPAD, H)
    hb, als, ald, scon = _tc_mid(acc, b1, W2, as2, ad2)
    acc = _edge_kernel()(csrc, cdst, cnt, _as_words(hb), als, ald, scon
                         ).reshape(NPAD, H)
    hb, als, ald, scon = _tc_mid(acc, b2, W3, as3, ad3)
    acc = _edge_kernel()(csrc, cdst, cnt, _as_words(hb), als, ald, scon
                         ).reshape(NPAD, H)
    return _tc_final(acc, b3, Wl, bl)


# speedup vs baseline: 1.5464x; 1.5464x over previous
import functools

import jax
import jax.numpy as jnp
from jax import lax
from jax.experimental import pallas as pl
from jax.experimental.pallas import tpu as pltpu
from jax.experimental.pallas import tpu_sc as plsc

N = 10000
E = 320000
H = 128
D_OUT = 64
NEG_SLOPE = 0.2

NC = 2
NS = 16
NW = NC * NS
RPT = 320
NPAD = NW * RPT
CAP = 11520
CHUNK = 128
ECHUNK = 4000


def _tc_first_body(x_ref, w_ref, as_ref, ad_ref, hb_ref, als_ref, ald_ref, sc_ref):
    h = jnp.dot(x_ref[...], w_ref[...], preferred_element_type=jnp.float32)
    hb_ref[...] = h.astype(jnp.bfloat16)
    als = jnp.sum(h * as_ref[...], axis=1)
    ald = jnp.sum(h * ad_ref[...], axis=1)
    als_ref[...] = als
    ald_ref[...] = ald
    sc_ref[...] = jnp.full((16,), jnp.max(als), jnp.float32)


def _tc_first(x, w, a_src, a_dst):
    return pl.pallas_call(
        _tc_first_body,
        out_shape=(
            jax.ShapeDtypeStruct((NPAD, H), jnp.bfloat16),
            jax.ShapeDtypeStruct((NPAD,), jnp.float32),
            jax.ShapeDtypeStruct((NPAD,), jnp.float32),
            jax.ShapeDtypeStruct((16,), jnp.float32),
        ),
    )(x, w, a_src.reshape(1, H), a_dst.reshape(1, H))


def _tc_mid_body(acc_ref, b_ref, w_ref, as_ref, ad_ref,
                 hb_ref, als_ref, ald_ref, sc_ref):
    x = jnp.maximum(acc_ref[...] + b_ref[...], 0.0)
    h = jnp.dot(x, w_ref[...], preferred_element_type=jnp.float32)
    hb_ref[...] = h.astype(jnp.bfloat16)
    als = jnp.sum(h * as_ref[...], axis=1)
    ald = jnp.sum(h * ad_ref[...], axis=1)
    als_ref[...] = als
    ald_ref[...] = ald
    sc_ref[...] = jnp.full((16,), jnp.max(als), jnp.float32)


def _tc_mid(acc, b, w, a_src, a_dst):
    return pl.pallas_call(
        _tc_mid_body,
        out_shape=(
            jax.ShapeDtypeStruct((NPAD, H), jnp.bfloat16),
            jax.ShapeDtypeStruct((NPAD,), jnp.float32),
            jax.ShapeDtypeStruct((NPAD,), jnp.float32),
            jax.ShapeDtypeStruct((16,), jnp.float32),
        ),
    )(acc, b.reshape(1, H), w, a_src.reshape(1, H), a_dst.reshape(1, H))


def _tc_final_body(acc_ref, b_ref, w_ref, bl_ref, o_ref):
    x = jnp.maximum(acc_ref[...] + b_ref[...], 0.0)
    out = jnp.dot(x, w_ref[...], preferred_element_type=jnp.float32)
    o_ref[...] = out[:N] + bl_ref[...]


def _tc_final(acc, b, wl, bl):
    return pl.pallas_call(
        _tc_final_body,
        out_shape=jax.ShapeDtypeStruct((N, D_OUT), jnp.float32),
    )(acc, b.reshape(1, H), wl, bl.reshape(1, D_OUT))


def _worker_id():
    return lax.axis_index("s") * NC + lax.axis_index("c")


NBUF_PF = 4


def _prefilter_body(src_hbm, dst_hbm, csrc_hbm, cdst_hbm, cnt_hbm,
                    src_v0, src_v1, src_v2, src_v3,
                    dst_v0, dst_v1, dst_v2, dst_v3, csrc_v, cdst_v, cnt_v,
                    sems_s, sems_d):
    src_v = (src_v0, src_v1, src_v2, src_v3)
    dst_v = (dst_v0, dst_v1, dst_v2, dst_v3)
    wid = _worker_id()
    lo = wid * RPT
    lov = jnp.full((16,), lo, jnp.int32)
    hiv = jnp.full((16,), lo + RPT, jnp.int32)
    trash = lax.iota(jnp.int32, 16) + jnp.full((16,), CAP - 16, jnp.int32)
    nchunks = E // ECHUNK

    def issue(i, b):
        off = i * ECHUNK
        pltpu.async_copy(src_hbm.at[pl.ds(off, ECHUNK)], src_v[b], sems_s.at[b])
        pltpu.async_copy(dst_hbm.at[pl.ds(off, ECHUNK)], dst_v[b], sems_d.at[b])

    def waitb(b):
        pltpu.make_async_copy(src_hbm.at[pl.ds(0, ECHUNK)], src_v[b],
                              sems_s.at[b]).wait()
        pltpu.make_async_copy(dst_hbm.at[pl.ds(0, ECHUNK)], dst_v[b],
                              sems_d.at[b]).wait()

    for b in range(NBUF_PF):
        issue(b, b)

    ones = jnp.full((16,), 1, jnp.int32)
    zeros16 = jnp.full((16,), 0, jnp.int32)
    capv = jnp.full((16,), CAP - 176, jnp.int32)

    def grp(s16, d16, nvec):
        msk = (d16 >= lov) & (d16 < hiv)
        mi = jnp.where(msk, ones, zeros16)
        inc = plsc.cumsum(mi)
        pcv = plsc.all_reduce_population_count(msk)
        pos = jnp.where(msk, inc + nvec - ones, trash)
        plsc.store_scatter(csrc_v, [pos], s16)
        plsc.store_scatter(cdst_v, [pos], d16 - lov)
        return jnp.minimum(nvec + pcv, capv)

    def outer(J, n):
        for b in range(NBUF_PF):
            i = J * NBUF_PF + b
            waitb(b)

            def grp2(g, n, b=b):
                n = grp(src_v[b][pl.ds(g * 32, 16)],
                        dst_v[b][pl.ds(g * 32, 16)], n)
                n = grp(src_v[b][pl.ds(g * 32 + 16, 16)],
                        dst_v[b][pl.ds(g * 32 + 16, 16)], n)
                return n

            n = lax.fori_loop(0, ECHUNK // 32, grp2, n)

            @pl.when(i + NBUF_PF < nchunks)
            def _(i=i, b=b):
                issue(i + NBUF_PF, b)
        return n

    nvec = lax.fori_loop(0, nchunks // NBUF_PF, outer,
                         jnp.full((16,), 0, jnp.int32))
    n = nvec[0]

    zeros = jnp.zeros((16,), jnp.int32)

    def zb(k, _):
        csrc_v[pl.ds(n + k * 16, 16)] = zeros
        cdst_v[pl.ds(n + k * 16, 16)] = zeros
        return 0

    lax.fori_loop(0, (CHUNK + 16) // 16, zb, 0)

    cnt_v[...] = jnp.full((16,), n, jnp.int32)
    pltpu.sync_copy(csrc_v, csrc_hbm.at[wid])
    pltpu.sync_copy(cdst_v, cdst_hbm.at[wid])
    pltpu.sync_copy(cnt_v, cnt_hbm.at[wid])


@functools.cache
def _prefilter_kernel():
    mesh = plsc.VectorSubcoreMesh(
        core_axis_name="c", subcore_axis_name="s", num_cores=NC, num_subcores=NS)
    return pl.kernel(
        _prefilter_body,
        out_type=(
            jax.ShapeDtypeStruct((NW, CAP), jnp.int32),
            jax.ShapeDtypeStruct((NW, CAP), jnp.int32),
            jax.ShapeDtypeStruct((NW, 16), jnp.int32),
        ),
        mesh=mesh,
        compiler_params=pltpu.CompilerParams(needs_layout_passes=False),
        scratch_types=(
            [pltpu.VMEM((ECHUNK,), jnp.int32)] * (2 * NBUF_PF)
            + [
                pltpu.VMEM((CAP,), jnp.int32),
                pltpu.VMEM((CAP,), jnp.int32),
                pltpu.VMEM((16,), jnp.int32),
                pltpu.SemaphoreType.DMA((NBUF_PF,)),
                pltpu.SemaphoreType.DMA((NBUF_PF,)),
            ]
        ),
    )


NBUF = 5


def _edge_body(csrc_hbm, cdst_hbm, cnt_hbm, h_hbm, als_hbm, ald_hbm, scon_hbm,
               acc_hbm,
               csrc_v, cdst_v, cnt_v, als_v, ald_v, scon_v, rows_v,
               acc_v, den_v, sems):
    wid = _worker_id()
    lo = wid * RPT

    pltpu.sync_copy(csrc_hbm.at[wid], csrc_v)
    pltpu.sync_copy(cdst_hbm.at[wid], cdst_v)
    pltpu.sync_copy(cnt_hbm.at[wid], cnt_v)
    pltpu.sync_copy(als_hbm, als_v)
    pltpu.sync_copy(ald_hbm.at[pl.ds(lo, RPT)], ald_v)
    pltpu.sync_copy(scon_hbm, scon_v)

    zrow = jnp.zeros((16,), jnp.float32)

    def zacc(i, _):
        for u in range(4):
            acc_v[pl.ds((i * 4 + u) * 16, 16)] = zrow
        return 0

    lax.fori_loop(0, RPT * 8 // 4, zacc, 0)

    def zden(i, _):
        den_v[pl.ds(i * 16, 16)] = zrow
        return 0

    lax.fori_loop(0, (RPT + 16) // 16, zden, 0)

    n_e = cnt_v[pl.ds(0, 16)][0]
    nchunks = (n_e + CHUNK - 1) // CHUNK
    Sv = scon_v[pl.ds(0, 16)]
    iota = lax.iota(jnp.int32, 16)
    lane0 = iota == 0
    den_trash = RPT + iota
    cols = [iota + 16 * r for r in range(8)]
    kidxs = [jnp.full((16,), k, jnp.int32) for k in range(16)]
    h128 = jnp.full((16,), H, jnp.int32)
    iota2 = iota * 2
    cols_e = [iota2 + 32 * q for q in range(4)]
    cols_o = [iota2 + 32 * q + 1 for q in range(4)]

    def bcast(v, k):
        return v.at[kidxs[k]].get(mode="promise_in_bounds")

    hw_hbm = h_hbm

    def issue(j, b):
        pltpu.async_copy(
            hw_hbm.at[csrc_v.at[pl.ds(j * CHUNK, CHUNK)]],
            rows_v.at[b], sems.at[b])

    def waitb(b):
        pltpu.make_async_copy(
            hw_hbm.at[csrc_v.at[pl.ds(0, CHUNK)]],
            rows_v.at[b], sems.at[b]).wait()

    for b in range(NBUF):
        @pl.when(b < nchunks)
        def _(b=b):
            issue(b, b)

    def outer(J, _):
        for b in range(NBUF):
            j = J * NBUF + b

            @pl.when(j < nchunks)
            def _(j=j, b=b):
                base = j * CHUNK
                waitb(b)

                def grp_body(g, _):
                    off = base + g * 16
                    s16 = csrc_v[pl.ds(off, 16)]
                    dl16 = cdst_v[pl.ds(off, 16)]
                    sa = plsc.load_gather(als_v, [s16])
                    da = plsc.load_gather(ald_v, [dl16])
                    e = sa + da
                    e = jnp.where(e > 0, e, NEG_SLOPE * e)
                    u = Sv + da
                    m = jnp.where(u > 0, u, NEG_SLOPE * u)
                    t = jnp.exp(e - m)
                    pos = iota + jnp.full((16,), off, jnp.int32)
                    nev = jnp.full((16,), n_e, jnp.int32)
                    t = jnp.where(pos < nev, t, jnp.zeros((16,), jnp.float32))
                    db16 = dl16 * h128
                    ei0 = g * 16

                    def stage(k):
                        tkv = bcast(t, k)
                        dbv = bcast(db16, k)
                        dnv = bcast(dl16, k)
                        halves = []
                        for q in range(4):
                            w16 = rows_v[b, ei0 + k, pl.ds(16 * q, 16)]
                            ab = plsc.bitcast(w16, jnp.bfloat16)
                            halves.append(plsc.unpack(
                                ab, format=plsc.PackFormat.INTERLEAVED))
                        return tkv, dbv, dnv, halves

                    def drain(st):
                        tkv, dbv, dnv, halves = st
                        for q in range(4):
                            ae, ao = halves[q]
                            plsc.addupdate_scatter(
                                acc_v, [dbv + cols_e[q]], ae * tkv)
                            plsc.addupdate_scatter(
                                acc_v, [dbv + cols_o[q]], ao * tkv)
                        didx = jnp.where(lane0, dnv, den_trash)
                        plsc.addupdate_scatter(den_v, [didx], tkv)

                    prev = stage(0)
                    for k in range(1, 16):
                        cur = stage(k)
                        drain(prev)
                        prev = cur
                    drain(prev)
                    return 0

                lax.fori_loop(0, CHUNK // 16, grp_body, 0)

                @pl.when(j + NBUF < nchunks)
                def _(j=j, b=b):
                    issue(j + NBUF, b)
        return 0

    lax.fori_loop(0, (nchunks + NBUF - 1) // NBUF, outer, 0)

    def norm(i, _):
        dv = den_v[pl.ds(i * 16, 16)]
        rv = 1.0 / (dv + 1e-16)
        for k in range(16):
            rkv = bcast(rv, k)
            base = (i * 16 + k) * H
            for r in range(8):
                acc_v[pl.ds(base + 16 * r, 16)] = (
                    acc_v[pl.ds(base + 16 * r, 16)] * rkv)
        return 0

    lax.fori_loop(0, RPT // 16, norm, 0)

    pltpu.sync_copy(acc_v, acc_hbm.at[pl.ds(lo * H, RPT * H)])


@functools.cache
def _edge_kernel():
    mesh = plsc.VectorSubcoreMesh(
        core_axis_name="c", subcore_axis_name="s", num_cores=NC, num_subcores=NS)
    return pl.kernel(
        _edge_body,
        out_type=jax.ShapeDtypeStruct((NPAD * H,), jnp.float32),
        mesh=mesh,
        compiler_params=pltpu.CompilerParams(
            needs_layout_passes=False, use_tc_tiling_on_sc=False),
        scratch_types=[
            pltpu.VMEM((CAP,), jnp.int32),
            pltpu.VMEM((CAP,), jnp.int32),
            pltpu.VMEM((16,), jnp.int32),
            pltpu.VMEM((NPAD,), jnp.float32),
            pltpu.VMEM((RPT,), jnp.float32),
            pltpu.VMEM((16,), jnp.float32),
            pltpu.VMEM((NBUF, CHUNK, H // 2), jnp.int32),
            pltpu.VMEM((RPT * H,), jnp.float32),
            pltpu.VMEM((RPT + 16,), jnp.float32),
            pltpu.SemaphoreType.DMA((NBUF,)),
        ],
    )


def kernel(x, edge_index, W1, as1, ad1, b1, W2, as2, ad2, b2, W3, as3, ad3, b3, Wl, bl):
    src = edge_index[0]
    dst = edge_index[1]
    xp = jnp.zeros((NPAD, H), jnp.float32).at[:N].set(x)

    csrc, cdst, cnt = _prefilter_kernel()(src, dst)

    def _as_words(hb):
        return lax.bitcast_convert_type(
            hb.reshape(NPAD, H // 2, 2), jnp.int32)

    hb, als, ald, scon = _tc_first(xp, W1, as1, ad1)
    acc = _edge_kernel()(csrc, cdst, cnt, _as_words(hb), als, ald, scon
                         ).reshape(NPAD, H)
    hb, als, ald, scon = _tc_mid(acc, b1, W2, as2, ad2)
    acc = _edge_kernel()(csrc, cdst, cnt, _as_words(hb), als, ald, scon
                         ).reshape(NPAD, H)
    hb, als, ald, scon = _tc_mid(acc, b2, W3, as3, ad3)
    acc = _edge_kernel()(csrc, cdst, cnt, _as_words(hb), als, ald, scon
                         ).reshape(NPAD, H)
    return _tc_final(acc, b3, Wl, bl)
